# Initial kernel scaffold; baseline (speedup 1.0000x reference)
import jax
import jax.numpy as jnp
from jax.experimental import pallas as pl


def kernel(x_user, x_movie, edge_index, edge_label_index, W1l_um, b1l_um, W1r_um, W1l_mu, b1l_mu, W1r_mu, W2l_um, b2l_um, W2r_um, W2l_mu, b2l_mu, W2r_mu, Wlin, blin):
    raise NotImplementedError("write your pallas kernel here")


# speedup vs baseline: 6.6541x; 6.6541x over previous
import functools

import jax
import jax.numpy as jnp
from jax import lax
from jax.experimental import pallas as pl
from jax.experimental.pallas import tpu as pltpu
from jax.experimental.pallas import tpu_sc as plsc

NC, NS = 2, 16
NW = NC * NS
N = 5000
NPAD = 5120
RPT = NPAD // NS
E = 640000
EPW = E // NW
KE = 80
L = 100000
LPAD = 100352
LPW = LPAD // NW
KL = 112
D = 128
CW = 16

_mesh = plsc.VectorSubcoreMesh(
    core_axis_name="c", subcore_axis_name="s", num_cores=NC, num_subcores=NS)


def _worker_id():
    return lax.axis_index("s") * NC + lax.axis_index("c")


def _segsum_counts_body(tab_u, tab_m, iu_hbm, im_hbm, zb, zs, ones_hbm,
                        out_m, out_u, cnt_m, cnt_u,
                        acc_m, acc_u, cm, cu, iu_v, im_v, rows_u, rows_m,
                        ones_v, cv, sem):
    cid = lax.axis_index("c")
    tid = lax.axis_index("s")
    r0 = tid * RPT
    pltpu.sync_copy(zb.at[pl.ds(r0, RPT)], acc_m.at[pl.ds(r0, RPT)])
    pltpu.sync_copy(zb.at[pl.ds(r0, RPT)], acc_u.at[pl.ds(r0, RPT)])
    pltpu.sync_copy(zs.at[pl.ds(r0, RPT)], cv)
    pltpu.sync_copy(cv, cm.at[pl.ds(r0, RPT)])
    pltpu.sync_copy(cv, cu.at[pl.ds(r0, RPT)])
    pltpu.sync_copy(ones_hbm, ones_v)
    plsc.subcore_barrier()
    base = _worker_id() * EPW

    def body(ci, carry):
        off = base + ci * KE
        pltpu.sync_copy(iu_hbm.at[pl.ds(off, KE)], iu_v)
        pltpu.sync_copy(im_hbm.at[pl.ds(off, KE)], im_v)
        pltpu.async_copy(tab_u.at[iu_v], rows_u, sem).wait()
        pltpu.sync_copy(rows_u, acc_m.at[im_v], add=True)
        pltpu.async_copy(tab_m.at[im_v], rows_m, sem).wait()
        pltpu.sync_copy(rows_m, acc_u.at[iu_v], add=True)
        pltpu.sync_copy(ones_v, cm.at[im_v], add=True)
        pltpu.sync_copy(ones_v, cu.at[iu_v], add=True)
        return carry

    lax.fori_loop(0, EPW // KE, body, 0)
    plsc.subcore_barrier()
    pltpu.sync_copy(acc_m.at[pl.ds(r0, RPT)], out_m.at[cid, pl.ds(r0, RPT)])
    pltpu.sync_copy(acc_u.at[pl.ds(r0, RPT)], out_u.at[cid, pl.ds(r0, RPT)])
    pltpu.sync_copy(cm.at[pl.ds(r0, RPT)], cv)
    pltpu.sync_copy(cv, cnt_m.at[pl.ds(cid * NPAD + r0, RPT)])
    pltpu.sync_copy(cu.at[pl.ds(r0, RPT)], cv)
    pltpu.sync_copy(cv, cnt_u.at[pl.ds(cid * NPAD + r0, RPT)])


def _segsum_body(tab_u, tab_m, iu_hbm, im_hbm, zb,
                 out_m, out_u,
                 acc_m, acc_u, iu_v, im_v, rows_u, rows_m, sem):
    cid = lax.axis_index("c")
    tid = lax.axis_index("s")
    r0 = tid * RPT
    pltpu.sync_copy(zb.at[pl.ds(r0, RPT)], acc_m.at[pl.ds(r0, RPT)])
    pltpu.sync_copy(zb.at[pl.ds(r0, RPT)], acc_u.at[pl.ds(r0, RPT)])
    plsc.subcore_barrier()
    base = _worker_id() * EPW

    def body(ci, carry):
        off = base + ci * KE
        pltpu.sync_copy(iu_hbm.at[pl.ds(off, KE)], iu_v)
        pltpu.sync_copy(im_hbm.at[pl.ds(off, KE)], im_v)
        pltpu.async_copy(tab_u.at[iu_v], rows_u, sem).wait()
        pltpu.sync_copy(rows_u, acc_m.at[im_v], add=True)
        pltpu.async_copy(tab_m.at[im_v], rows_m, sem).wait()
        pltpu.sync_copy(rows_m, acc_u.at[iu_v], add=True)
        return carry

    lax.fori_loop(0, EPW // KE, body, 0)
    plsc.subcore_barrier()
    pltpu.sync_copy(acc_m.at[pl.ds(r0, RPT)], out_m.at[cid, pl.ds(r0, RPT)])
    pltpu.sync_copy(acc_u.at[pl.ds(r0, RPT)], out_u.at[cid, pl.ds(r0, RPT)])


_f32 = jnp.float32
_segsum_counts = pl.kernel(
    _segsum_counts_body,
    out_type=(jax.ShapeDtypeStruct((NC, NPAD, D), _f32),
              jax.ShapeDtypeStruct((NC, NPAD, D), _f32),
              jax.ShapeDtypeStruct((NC * NPAD,), _f32),
              jax.ShapeDtypeStruct((NC * NPAD,), _f32)),
    mesh=_mesh,
    scratch_types=[
        pltpu.VMEM_SHARED((NPAD, D), _f32),
        pltpu.VMEM_SHARED((NPAD, D), _f32),
        pltpu.VMEM_SHARED((NPAD,), _f32),
        pltpu.VMEM_SHARED((NPAD,), _f32),
        pltpu.VMEM((KE,), jnp.int32),
        pltpu.VMEM((KE,), jnp.int32),
        pltpu.VMEM((KE, D), _f32),
        pltpu.VMEM((KE, D), _f32),
        pltpu.VMEM((KE,), _f32),
        pltpu.VMEM((RPT,), _f32),
        pltpu.SemaphoreType.DMA,
    ],
)

_segsum = pl.kernel(
    _segsum_body,
    out_type=(jax.ShapeDtypeStruct((NC, NPAD, D), _f32),
              jax.ShapeDtypeStruct((NC, NPAD, D), _f32)),
    mesh=_mesh,
    scratch_types=[
        pltpu.VMEM_SHARED((NPAD, D), _f32),
        pltpu.VMEM_SHARED((NPAD, D), _f32),
        pltpu.VMEM((KE,), jnp.int32),
        pltpu.VMEM((KE,), jnp.int32),
        pltpu.VMEM((KE, D), _f32),
        pltpu.VMEM((KE, D), _f32),
        pltpu.SemaphoreType.DMA,
    ],
)


def _label_gather_body(pu_hbm, pm_hbm, row_hbm, col_hbm,
                       gu_o, gm_o, ri_v, ci_v, ru_v, rm_v, sem):
    base = _worker_id() * LPW

    def body(ci, carry):
        off = base + ci * KL
        pltpu.sync_copy(row_hbm.at[pl.ds(off, KL)], ri_v)
        pltpu.sync_copy(col_hbm.at[pl.ds(off, KL)], ci_v)
        pltpu.async_copy(pu_hbm.at[ri_v], ru_v, sem).wait()
        pltpu.async_copy(pm_hbm.at[ci_v], rm_v, sem).wait()
        pltpu.sync_copy(ru_v, gu_o.at[pl.ds(off, KL)])
        pltpu.sync_copy(rm_v, gm_o.at[pl.ds(off, KL)])
        return carry

    lax.fori_loop(0, LPW // KL, body, 0)


_label_gather = pl.kernel(
    _label_gather_body,
    out_type=(jax.ShapeDtypeStruct((LPAD, D), _f32),
              jax.ShapeDtypeStruct((LPAD, D), _f32)),
    mesh=_mesh,
    scratch_types=[
        pltpu.VMEM((KL,), jnp.int32),
        pltpu.VMEM((KL,), jnp.int32),
        pltpu.VMEM((KL, D), _f32),
        pltpu.VMEM((KL, D), _f32),
        pltpu.SemaphoreType.DMA,
    ],
)


_BR = 512


def _sage_half(acc, cnt, x, Wl, b, Wr):
    a = acc[...]
    s = a[0] + a[1]
    c = cnt[...]
    ctot = c[0] + c[1]
    inv = 1.0 / jnp.maximum(ctot, 1.0)
    mean = s * inv[:, None]
    return (jnp.dot(mean, Wl[...], preferred_element_type=jnp.float32)
            + b[...]
            + jnp.dot(x[...], Wr[...], preferred_element_type=jnp.float32))


def _tc_layer1_body(acc_m, cnt_m, xm, WlA, bA, WrA,
                    acc_u, cnt_u, xu, WlB, bB, WrB, hm_o, hu_o):
    hm_o[...] = jnp.maximum(_sage_half(acc_m, cnt_m, xm, WlA, bA, WrA), 0.0)
    hu_o[...] = jnp.maximum(_sage_half(acc_u, cnt_u, xu, WlB, bB, WrB), 0.0)


def _tc_layer2_body(acc_m, cnt_m, xm, WlA, bA, WrA,
                    acc_u, cnt_u, xu, WlB, bB, WrB,
                    WpU, bpU, WpM, pu_o, pm_o):
    zm = _sage_half(acc_m, cnt_m, xm, WlA, bA, WrA)
    zu = _sage_half(acc_u, cnt_u, xu, WlB, bB, WrB)
    pu_o[...] = jnp.dot(zu, WpU[...], preferred_element_type=jnp.float32) + bpU[...]
    pm_o[...] = jnp.dot(zm, WpM[...], preferred_element_type=jnp.float32)


def _acc_spec():
    return pl.BlockSpec((NC, _BR, D), lambda i: (0, i, 0))


def _cnt_spec():
    return pl.BlockSpec((NC, _BR), lambda i: (0, i))


def _row_spec(w=D):
    return pl.BlockSpec((_BR, w), lambda i: (i, 0))


def _w_spec(w=D):
    return pl.BlockSpec((D, w), lambda i: (0, 0))


def _b_spec(w=D):
    return pl.BlockSpec((1, w), lambda i: (0, 0))


_tc_layer1 = pl.pallas_call(
    _tc_layer1_body,
    grid=(NPAD // _BR,),
    in_specs=[_acc_spec(), _cnt_spec(), _row_spec(), _w_spec(), _b_spec(), _w_spec(),
              _acc_spec(), _cnt_spec(), _row_spec(), _w_spec(), _b_spec(), _w_spec()],
    out_specs=(_row_spec(), _row_spec()),
    out_shape=(jax.ShapeDtypeStruct((NPAD, D), _f32),
               jax.ShapeDtypeStruct((NPAD, D), _f32)),
)

_tc_layer2 = pl.pallas_call(
    _tc_layer2_body,
    grid=(NPAD // _BR,),
    in_specs=[_acc_spec(), _cnt_spec(), _row_spec(), _w_spec(), _b_spec(), _w_spec(),
              _acc_spec(), _cnt_spec(), _row_spec(), _w_spec(), _b_spec(), _w_spec(),
              _w_spec(D), _b_spec(D), _w_spec(D)],
    out_specs=(_row_spec(D), _row_spec(D)),
    out_shape=(jax.ShapeDtypeStruct((NPAD, D), _f32),
               jax.ShapeDtypeStruct((NPAD, D), _f32)),
)


def _head_body(gu, gm, o):
    s = gu[...] + gm[...]
    colid = lax.broadcasted_iota(jnp.int32, s.shape, 1)
    o[...] = jnp.where(colid == 1, jax.nn.softplus(s) + 1e-6, s)


_BL = 1024
_head = pl.pallas_call(
    _head_body,
    grid=(LPAD // _BL,),
    in_specs=[pl.BlockSpec((_BL, D), lambda i: (i, 0)),
              pl.BlockSpec((_BL, D), lambda i: (i, 0))],
    out_specs=pl.BlockSpec((_BL, D), lambda i: (i, 0)),
    out_shape=jax.ShapeDtypeStruct((LPAD, D), _f32),
)


def kernel(x_user, x_movie, edge_index, edge_label_index,
           W1l_um, b1l_um, W1r_um, W1l_mu, b1l_mu, W1r_mu,
           W2l_um, b2l_um, W2r_um, W2l_mu, b2l_mu, W2r_mu, Wlin, blin):
    f32 = jnp.float32
    pad_n = NPAD - N
    xu = jnp.pad(x_user.astype(f32), ((0, pad_n), (0, 0)))
    xm = jnp.pad(x_movie.astype(f32), ((0, pad_n), (0, 0)))
    iu = edge_index[0].astype(jnp.int32)
    im = edge_index[1].astype(jnp.int32)
    row = jnp.pad(edge_label_index[0].astype(jnp.int32), (0, LPAD - L))
    col = jnp.pad(edge_label_index[1].astype(jnp.int32), (0, LPAD - L))

    zb = jnp.zeros((NPAD, D), f32)
    zs = jnp.zeros((NPAD,), f32)
    ones = jnp.ones((KE,), f32)

    acc_m, acc_u, cnt_m, cnt_u = _segsum_counts(xu, xm, iu, im, zb, zs, ones)
    cnt_m = cnt_m.reshape(NC, NPAD)
    cnt_u = cnt_u.reshape(NC, NPAD)

    b1um = b1l_um.reshape(1, D).astype(f32)
    b1mu = b1l_mu.reshape(1, D).astype(f32)
    h_movie, h_user = _tc_layer1(acc_m, cnt_m, xm, W1l_um, b1um, W1r_um,
                                 acc_u, cnt_u, xu, W1l_mu, b1mu, W1r_mu)

    acc2_m, acc2_u = _segsum(h_user, h_movie, iu, im, zb)

    WpU = jnp.zeros((D, D), f32).at[:, 0:2].set(Wlin[:D].astype(f32))
    WpM = jnp.zeros((D, D), f32).at[:, 0:2].set(Wlin[D:].astype(f32))
    bp = jnp.zeros((1, D), f32).at[0, 0:2].set(blin.astype(f32))
    b2um = b2l_um.reshape(1, D).astype(f32)
    b2mu = b2l_mu.reshape(1, D).astype(f32)
    p_user, p_movie = _tc_layer2(acc2_m, cnt_m, h_movie, W2l_um, b2um, W2r_um,
                                 acc2_u, cnt_u, h_user, W2l_mu, b2mu, W2r_mu,
                                 WpU, bp, WpM)

    gu, gm = _label_gather(p_user, p_movie, row, col)
    out = _head(gu, gm)
    return out[:L, 0], out[:L, 1]


# speedup vs baseline: 10.8078x; 1.6242x over previous
import jax
import jax.numpy as jnp
from jax import lax
from jax.experimental import pallas as pl
from jax.experimental.pallas import tpu as pltpu
from jax.experimental.pallas import tpu_sc as plsc

NC, NS = 2, 16
NW = NC * NS
N = 5000
NPAD = 5120
RPT = NPAD // NS
E = 640000
EPW = E // NW
KE = 80
NCH = EPW // KE
L = 100000
LPAD = 100352
LPW = LPAD // NW
KL = 112
NCL = LPW // KL
D = 128
_f32 = jnp.float32

_mesh = plsc.VectorSubcoreMesh(
    core_axis_name="c", subcore_axis_name="s", num_cores=NC, num_subcores=NS)


def _worker_id():
    return lax.axis_index("s") * NC + lax.axis_index("c")


EPT = E // NS
EPH = EPT // 2
NCH2 = EPH // KE


def _make_segsum_body(with_counts):
    def body(*refs):
        if with_counts:
            (tab_u, tab_m, iu_hbm, im_hbm, zb, zs, ones_hbm,
             out_m, out_u, cnt_m, cnt_u,
             acc, cnt, gi_all, si_all,
             gi0, gi1, si0, si1, r0b, r1b, ones_v, cv,
             sg0, sg1, ss0, ss1, sc0, sc1) = refs
            s_c = (sc0, sc1)
        else:
            (tab_u, tab_m, iu_hbm, im_hbm, zb,
             out_m, out_u,
             acc, gi_all, si_all,
             gi0, gi1, si0, si1, r0b, r1b,
             sg0, sg1, ss0, ss1) = refs
            cnt = None
        gi_w, si_w = (gi0, gi1), (si0, si1)
        rw = (r0b, r1b)
        s_g, s_s = (sg0, sg1), (ss0, ss1)

        cid = lax.axis_index("c")
        tid = lax.axis_index("s")
        r0 = tid * RPT
        pltpu.sync_copy(zb.at[pl.ds(r0, RPT)], acc.at[pl.ds(r0, RPT)])
        if with_counts:
            pltpu.sync_copy(zs.at[pl.ds(r0, RPT)], cv)
            pltpu.sync_copy(cv, cnt.at[pl.ds(r0, RPT)])
            pltpu.sync_copy(ones_hbm, ones_v)
        def run_dir(tab, gidx_hbm, sidx_hbm, out_a, cnt_o):
            def step(c, b, first):
                giv, siv = gi_w[b], si_w[b]
                rb = rw[b]
                if not first:
                    pltpu.make_async_copy(rb, acc.at[siv], s_s[b]).wait()
                    if with_counts:
                        pltpu.make_async_copy(ones_v, cnt.at[siv], s_c[b]).wait()
                off = c * KE
                for i in range(KE // 16):
                    giv[pl.ds(i * 16, 16)] = gi_all[pl.ds(off + i * 16, 16)]
                    siv[pl.ds(i * 16, 16)] = si_all[pl.ds(off + i * 16, 16)]
                pltpu.async_copy(tab.at[giv], rb, s_g[b]).wait()
                pltpu.async_copy(rb, acc.at[siv], s_s[b], add=True)
                if with_counts:
                    pltpu.async_copy(ones_v, cnt.at[siv], s_c[b], add=True)

            for h in (0, 1):
                base = tid * EPT + h * EPH
                pltpu.sync_copy(gidx_hbm.at[pl.ds(base, EPH)], gi_all)
                pltpu.sync_copy(sidx_hbm.at[pl.ds(base, EPH)], si_all)
                if h == 0:
                    plsc.subcore_barrier()
                step(0, 0, True)
                step(1, 1, True)

                def pair(p, carry):
                    step(2 * p, 0, False)
                    step(2 * p + 1, 1, False)
                    return carry

                lax.fori_loop(1, NCH2 // 2, pair, 0)
                for b in (0, 1):
                    pltpu.make_async_copy(rw[b], acc.at[si_w[b]], s_s[b]).wait()
                    if with_counts:
                        pltpu.make_async_copy(ones_v, cnt.at[si_w[b]], s_c[b]).wait()
            plsc.subcore_barrier()
            pltpu.sync_copy(acc.at[pl.ds(r0, RPT)], out_a.at[pl.ds(r0, RPT)])
            if with_counts:
                pltpu.sync_copy(cnt.at[pl.ds(r0, RPT)], cv)
                pltpu.sync_copy(cv, cnt_o.at[pl.ds(r0, RPT)])

        @pl.when(cid == 0)
        def _():
            run_dir(tab_u, iu_hbm, im_hbm, out_m, cnt_m if with_counts else None)

        @pl.when(cid == 1)
        def _():
            run_dir(tab_m, im_hbm, iu_hbm, out_u, cnt_u if with_counts else None)

    return body


def _seg_scratch(with_counts):
    sc = [
        pltpu.VMEM_SHARED((NPAD, D), _f32),
        pltpu.VMEM_SHARED((NPAD,), _f32) if with_counts else None,
        pltpu.VMEM((EPH,), jnp.int32),
        pltpu.VMEM((EPH,), jnp.int32),
        pltpu.VMEM((KE,), jnp.int32),
        pltpu.VMEM((KE,), jnp.int32),
        pltpu.VMEM((KE,), jnp.int32),
        pltpu.VMEM((KE,), jnp.int32),
        pltpu.VMEM((KE, D), _f32),
        pltpu.VMEM((KE, D), _f32),
    ]
    if with_counts:
        sc += [pltpu.VMEM((KE,), _f32), pltpu.VMEM((RPT,), _f32)]
    sc = [s for s in sc if s is not None]
    nsem = 6 if with_counts else 4
    sc += [pltpu.SemaphoreType.DMA] * nsem
    return sc


_segsum_counts = pl.kernel(
    _make_segsum_body(True),
    out_type=(jax.ShapeDtypeStruct((NPAD, D), _f32),
              jax.ShapeDtypeStruct((NPAD, D), _f32),
              jax.ShapeDtypeStruct((NPAD,), _f32),
              jax.ShapeDtypeStruct((NPAD,), _f32)),
    mesh=_mesh,
    scratch_types=_seg_scratch(True),
)

_segsum = pl.kernel(
    _make_segsum_body(False),
    out_type=(jax.ShapeDtypeStruct((NPAD, D), _f32),
              jax.ShapeDtypeStruct((NPAD, D), _f32)),
    mesh=_mesh,
    scratch_types=_seg_scratch(False),
)


def _label_gather_body(pu_hbm, pm_hbm, row_hbm, col_hbm,
                       gu_o, gm_o, ri_all, ci_all,
                       ru0, ru1, rm0, rm1, sgu0, sgu1, sgm0, sgm1,
                       swu0, swu1, swm0, swm1):
    ru_w, rm_w = (ru0, ru1), (rm0, rm1)
    s_g1, s_g2 = (sgu0, sgu1), (sgm0, sgm1)
    s_w1, s_w2 = (swu0, swu1), (swm0, swm1)
    base = _worker_id() * LPW
    pltpu.sync_copy(row_hbm.at[pl.ds(base, LPW)], ri_all)
    pltpu.sync_copy(col_hbm.at[pl.ds(base, LPW)], ci_all)

    def step(c, b, first):
        ru, rm = ru_w[b], rm_w[b]
        if not first:
            off2 = base + (c - 2) * KL
            pltpu.make_async_copy(ru, gu_o.at[pl.ds(off2, KL)], s_w1[b]).wait()
            pltpu.make_async_copy(rm, gm_o.at[pl.ds(off2, KL)], s_w2[b]).wait()
        loc = c * KL
        d1 = pltpu.async_copy(pu_hbm.at[ri_all.at[pl.ds(loc, KL)]], ru, s_g1[b])
        d2 = pltpu.async_copy(pm_hbm.at[ci_all.at[pl.ds(loc, KL)]], rm, s_g2[b])
        d1.wait()
        d2.wait()
        off = base + loc
        pltpu.async_copy(ru, gu_o.at[pl.ds(off, KL)], s_w1[b])
        pltpu.async_copy(rm, gm_o.at[pl.ds(off, KL)], s_w2[b])

    step(0, 0, True)
    step(1, 1, True)

    def pair(p, carry):
        step(2 * p, 0, False)
        step(2 * p + 1, 1, False)
        return carry

    lax.fori_loop(1, NCL // 2, pair, 0)
    for b in (0, 1):
        off2 = base + (NCL - 2 + b) * KL
        pltpu.make_async_copy(ru_w[b], gu_o.at[pl.ds(off2, KL)], s_w1[b]).wait()
        pltpu.make_async_copy(rm_w[b], gm_o.at[pl.ds(off2, KL)], s_w2[b]).wait()


_label_gather = pl.kernel(
    _label_gather_body,
    out_type=(jax.ShapeDtypeStruct((LPAD, D), _f32),
              jax.ShapeDtypeStruct((LPAD, D), _f32)),
    mesh=_mesh,
    scratch_types=[
        pltpu.VMEM((LPW,), jnp.int32),
        pltpu.VMEM((LPW,), jnp.int32),
        pltpu.VMEM((KL, D), _f32),
        pltpu.VMEM((KL, D), _f32),
        pltpu.VMEM((KL, D), _f32),
        pltpu.VMEM((KL, D), _f32),
    ] + [pltpu.SemaphoreType.DMA] * 8,
)


_BR = 512


def _sage_half(acc, cnt, x, Wl, b, Wr):
    s = acc[...]
    ctot = cnt[...]
    inv = 1.0 / jnp.maximum(ctot, 1.0)
    mean = s * inv[:, None]
    return (jnp.dot(mean, Wl[...], preferred_element_type=jnp.float32)
            + b[...]
            + jnp.dot(x[...], Wr[...], preferred_element_type=jnp.float32))


def _tc_layer1_body(acc_m, cnt_m, xm, WlA, bA, WrA,
                    acc_u, cnt_u, xu, WlB, bB, WrB, hm_o, hu_o):
    hm_o[...] = jnp.maximum(_sage_half(acc_m, cnt_m, xm, WlA, bA, WrA), 0.0)
    hu_o[...] = jnp.maximum(_sage_half(acc_u, cnt_u, xu, WlB, bB, WrB), 0.0)


def _tc_layer2_body(acc_m, cnt_m, xm, WlA, bA, WrA,
                    acc_u, cnt_u, xu, WlB, bB, WrB,
                    WpU, bpU, WpM, pu_o, pm_o):
    zm = _sage_half(acc_m, cnt_m, xm, WlA, bA, WrA)
    zu = _sage_half(acc_u, cnt_u, xu, WlB, bB, WrB)
    pu_o[...] = jnp.dot(zu, WpU[...], preferred_element_type=jnp.float32) + bpU[...]
    pm_o[...] = jnp.dot(zm, WpM[...], preferred_element_type=jnp.float32)


def _acc_spec():
    return pl.BlockSpec((_BR, D), lambda i: (i, 0))


def _cnt_spec():
    return pl.BlockSpec((_BR,), lambda i: (i,))


def _row_spec(w=D):
    return pl.BlockSpec((_BR, w), lambda i: (i, 0))


def _w_spec(w=D):
    return pl.BlockSpec((D, w), lambda i: (0, 0))


def _b_spec(w=D):
    return pl.BlockSpec((1, w), lambda i: (0, 0))


_tc_layer1 = pl.pallas_call(
    _tc_layer1_body,
    grid=(NPAD // _BR,),
    in_specs=[_acc_spec(), _cnt_spec(), _row_spec(), _w_spec(), _b_spec(), _w_spec(),
              _acc_spec(), _cnt_spec(), _row_spec(), _w_spec(), _b_spec(), _w_spec()],
    out_specs=(_row_spec(), _row_spec()),
    out_shape=(jax.ShapeDtypeStruct((NPAD, D), _f32),
               jax.ShapeDtypeStruct((NPAD, D), _f32)),
)

_tc_layer2 = pl.pallas_call(
    _tc_layer2_body,
    grid=(NPAD // _BR,),
    in_specs=[_acc_spec(), _cnt_spec(), _row_spec(), _w_spec(), _b_spec(), _w_spec(),
              _acc_spec(), _cnt_spec(), _row_spec(), _w_spec(), _b_spec(), _w_spec(),
              _w_spec(D), _b_spec(D), _w_spec(D)],
    out_specs=(_row_spec(D), _row_spec(D)),
    out_shape=(jax.ShapeDtypeStruct((NPAD, D), _f32),
               jax.ShapeDtypeStruct((NPAD, D), _f32)),
)


def _head_body(gu, gm, o):
    s = gu[...] + gm[...]
    colid = lax.broadcasted_iota(jnp.int32, s.shape, 1)
    o[...] = jnp.where(colid == 1, jax.nn.softplus(s) + 1e-6, s)


_BL = 1024
_head = pl.pallas_call(
    _head_body,
    grid=(LPAD // _BL,),
    in_specs=[pl.BlockSpec((_BL, D), lambda i: (i, 0)),
              pl.BlockSpec((_BL, D), lambda i: (i, 0))],
    out_specs=pl.BlockSpec((_BL, D), lambda i: (i, 0)),
    out_shape=jax.ShapeDtypeStruct((LPAD, D), _f32),
)


def kernel(x_user, x_movie, edge_index, edge_label_index,
           W1l_um, b1l_um, W1r_um, W1l_mu, b1l_mu, W1r_mu,
           W2l_um, b2l_um, W2r_um, W2l_mu, b2l_mu, W2r_mu, Wlin, blin):
    f32 = jnp.float32
    pad_n = NPAD - N
    xu = jnp.pad(x_user.astype(f32), ((0, pad_n), (0, 0)))
    xm = jnp.pad(x_movie.astype(f32), ((0, pad_n), (0, 0)))
    iu = edge_index[0].astype(jnp.int32)
    im = edge_index[1].astype(jnp.int32)
    row = jnp.pad(edge_label_index[0].astype(jnp.int32), (0, LPAD - L))
    col = jnp.pad(edge_label_index[1].astype(jnp.int32), (0, LPAD - L))

    zb = jnp.zeros((NPAD, D), f32)
    zs = jnp.zeros((NPAD,), f32)
    ones = jnp.ones((KE,), f32)

    acc_m, acc_u, cnt_m, cnt_u = _segsum_counts(xu, xm, iu, im, zb, zs, ones)

    b1um = b1l_um.reshape(1, D).astype(f32)
    b1mu = b1l_mu.reshape(1, D).astype(f32)
    h_movie, h_user = _tc_layer1(acc_m, cnt_m, xm, W1l_um, b1um, W1r_um,
                                 acc_u, cnt_u, xu, W1l_mu, b1mu, W1r_mu)

    acc2_m, acc2_u = _segsum(h_user, h_movie, iu, im, zb)

    WpU = jnp.zeros((D, D), f32).at[:, 0:2].set(Wlin[:D].astype(f32))
    WpM = jnp.zeros((D, D), f32).at[:, 0:2].set(Wlin[D:].astype(f32))
    bp = jnp.zeros((1, D), f32).at[0, 0:2].set(blin.astype(f32))
    b2um = b2l_um.reshape(1, D).astype(f32)
    b2mu = b2l_mu.reshape(1, D).astype(f32)
    p_user, p_movie = _tc_layer2(acc2_m, cnt_m, h_movie, W2l_um, b2um, W2r_um,
                                 acc2_u, cnt_u, h_user, W2l_mu, b2mu, W2r_mu,
                                 WpU, bp, WpM)

    gu, gm = _label_gather(p_user, p_movie, row, col)
    out = _head(gu, gm)
    return out[:L, 0], out[:L, 1]


# speedup vs baseline: 13.5484x; 1.2536x over previous
import jax
import jax.numpy as jnp
from jax import lax
from jax.experimental import pallas as pl
from jax.experimental.pallas import tpu as pltpu
from jax.experimental.pallas import tpu_sc as plsc

NC, NS = 2, 16
NW = NC * NS
N = 5000
NPAD = 5120
RPT = NPAD // NS
E = 640000
EPW = E // NW
KE = 80
NCH = EPW // KE
L = 100000
LPAD = 100352
LPW = LPAD // NW
KL = 112
NCL = LPW // KL
D = 128
_f32 = jnp.float32

_mesh = plsc.VectorSubcoreMesh(
    core_axis_name="c", subcore_axis_name="s", num_cores=NC, num_subcores=NS)


def _worker_id():
    return lax.axis_index("s") * NC + lax.axis_index("c")


EPT = E // NS
SEGS = ((0, 16000), (16000, 16000), (32000, 8000))
SEGMAX = 16000
MC = 2 * KE


def _make_segsum_body(with_counts):
    def body(*refs):
        if with_counts:
            (tab_u, tab_m, iu_hbm, im_hbm, zb, zs, ones_hbm,
             out_m, out_u, cnt_m, cnt_u,
             acc, cnt, gi_all, si_all,
             gi00, gi01, gi10, gi11, si00, si01, si10, si11,
             r00, r01, r10, r11, ones_v, cv,
             sg0, sg1, ss0, ss1, sc0, sc1) = refs
            s_c = (sc0, sc1)
        else:
            (tab_u, tab_m, iu_hbm, im_hbm, zb,
             out_m, out_u,
             acc, gi_all, si_all,
             gi00, gi01, gi10, gi11, si00, si01, si10, si11,
             r00, r01, r10, r11,
             sg0, sg1, ss0, ss1) = refs
            cnt = None
        gi_w = ((gi00, gi01), (gi10, gi11))
        si_w = ((si00, si01), (si10, si11))
        rw = ((r00, r01), (r10, r11))
        s_g, s_s = (sg0, sg1), (ss0, ss1)

        cid = lax.axis_index("c")
        tid = lax.axis_index("s")
        r0 = tid * RPT
        pltpu.sync_copy(zb.at[pl.ds(r0, RPT)], acc.at[pl.ds(r0, RPT)])
        if with_counts:
            pltpu.sync_copy(zs.at[pl.ds(r0, RPT)], cv)
            pltpu.sync_copy(cv, cnt.at[pl.ds(r0, RPT)])
            pltpu.sync_copy(ones_hbm, ones_v)

        def run_dir(tab, gidx_hbm, sidx_hbm, out_a, cnt_o):
            def step(c, b, first):
                if not first:
                    for sub in (0, 1):
                        pltpu.make_async_copy(
                            rw[b][sub], acc.at[si_w[b][sub]], s_s[b]).wait()
                        if with_counts:
                            pltpu.make_async_copy(
                                ones_v, cnt.at[si_w[b][sub]], s_c[b]).wait()
                for sub in (0, 1):
                    giv, siv = gi_w[b][sub], si_w[b][sub]
                    off = c * MC + sub * KE
                    for i in range(KE // 16):
                        giv[pl.ds(i * 16, 16)] = gi_all[pl.ds(off + i * 16, 16)]
                        siv[pl.ds(i * 16, 16)] = si_all[pl.ds(off + i * 16, 16)]
                d = [pltpu.async_copy(tab.at[gi_w[b][sub]], rw[b][sub], s_g[b])
                     for sub in (0, 1)]
                for sub in (0, 1):
                    d[sub].wait()
                    pltpu.async_copy(rw[b][sub], acc.at[si_w[b][sub]],
                                     s_s[b], add=True)
                    if with_counts:
                        pltpu.async_copy(ones_v, cnt.at[si_w[b][sub]],
                                         s_c[b], add=True)

            first_seg = True
            for soff, slen in SEGS:
                base = tid * EPT + soff
                pltpu.sync_copy(gidx_hbm.at[pl.ds(base, slen)],
                                gi_all.at[pl.ds(0, slen)])
                pltpu.sync_copy(sidx_hbm.at[pl.ds(base, slen)],
                                si_all.at[pl.ds(0, slen)])
                if first_seg:
                    plsc.subcore_barrier()
                step(0, 0, first_seg)
                step(1, 1, first_seg)

                def pair(p, carry):
                    step(2 * p, 0, False)
                    step(2 * p + 1, 1, False)
                    return carry

                lax.fori_loop(1, slen // MC // 2, pair, 0)
                first_seg = False
            for b in (0, 1):
                for sub in (0, 1):
                    pltpu.make_async_copy(
                        rw[b][sub], acc.at[si_w[b][sub]], s_s[b]).wait()
                    if with_counts:
                        pltpu.make_async_copy(
                            ones_v, cnt.at[si_w[b][sub]], s_c[b]).wait()
            plsc.subcore_barrier()
            pltpu.sync_copy(acc.at[pl.ds(r0, RPT)], out_a.at[pl.ds(r0, RPT)])
            if with_counts:
                pltpu.sync_copy(cnt.at[pl.ds(r0, RPT)], cv)
                pltpu.sync_copy(cv, cnt_o.at[pl.ds(r0, RPT)])

        @pl.when(cid == 0)
        def _():
            run_dir(tab_u, iu_hbm, im_hbm, out_m, cnt_m if with_counts else None)

        @pl.when(cid == 1)
        def _():
            run_dir(tab_m, im_hbm, iu_hbm, out_u, cnt_u if with_counts else None)

    return body


def _seg_scratch(with_counts):
    sc = [
        pltpu.VMEM_SHARED((NPAD, D), _f32),
        pltpu.VMEM_SHARED((NPAD,), _f32) if with_counts else None,
        pltpu.VMEM((SEGMAX,), jnp.int32),
        pltpu.VMEM((SEGMAX,), jnp.int32),
    ]
    sc += [pltpu.VMEM((KE,), jnp.int32) for _ in range(8)]
    sc += [pltpu.VMEM((KE, D), _f32) for _ in range(4)]
    if with_counts:
        sc += [pltpu.VMEM((KE,), _f32), pltpu.VMEM((RPT,), _f32)]
    sc = [s for s in sc if s is not None]
    nsem = 6 if with_counts else 4
    sc += [pltpu.SemaphoreType.DMA] * nsem
    return sc


_segsum_counts = pl.kernel(
    _make_segsum_body(True),
    out_type=(jax.ShapeDtypeStruct((NPAD, D), _f32),
              jax.ShapeDtypeStruct((NPAD, D), _f32),
              jax.ShapeDtypeStruct((NPAD,), _f32),
              jax.ShapeDtypeStruct((NPAD,), _f32)),
    mesh=_mesh,
    scratch_types=_seg_scratch(True),
)

_segsum = pl.kernel(
    _make_segsum_body(False),
    out_type=(jax.ShapeDtypeStruct((NPAD, D), _f32),
              jax.ShapeDtypeStruct((NPAD, D), _f32)),
    mesh=_mesh,
    scratch_types=_seg_scratch(False),
)


def _label_gather_body(pu_hbm, pm_hbm, row_hbm, col_hbm,
                       gu_o, gm_o, ri_all, ci_all,
                       ru0, ru1, rm0, rm1, sgu0, sgu1, sgm0, sgm1,
                       swu0, swu1, swm0, swm1):
    ru_w, rm_w = (ru0, ru1), (rm0, rm1)
    s_g1, s_g2 = (sgu0, sgu1), (sgm0, sgm1)
    s_w1, s_w2 = (swu0, swu1), (swm0, swm1)
    base = _worker_id() * LPW
    pltpu.sync_copy(row_hbm.at[pl.ds(base, LPW)], ri_all)
    pltpu.sync_copy(col_hbm.at[pl.ds(base, LPW)], ci_all)

    def step(c, b, first):
        ru, rm = ru_w[b], rm_w[b]
        if not first:
            off2 = base + (c - 2) * KL
            pltpu.make_async_copy(ru, gu_o.at[pl.ds(off2, KL)], s_w1[b]).wait()
            pltpu.make_async_copy(rm, gm_o.at[pl.ds(off2, KL)], s_w2[b]).wait()
        loc = c * KL
        d1 = pltpu.async_copy(pu_hbm.at[ri_all.at[pl.ds(loc, KL)]], ru, s_g1[b])
        d2 = pltpu.async_copy(pm_hbm.at[ci_all.at[pl.ds(loc, KL)]], rm, s_g2[b])
        d1.wait()
        d2.wait()
        off = base + loc
        pltpu.async_copy(ru, gu_o.at[pl.ds(off, KL)], s_w1[b])
        pltpu.async_copy(rm, gm_o.at[pl.ds(off, KL)], s_w2[b])

    step(0, 0, True)
    step(1, 1, True)

    def pair(p, carry):
        step(2 * p, 0, False)
        step(2 * p + 1, 1, False)
        return carry

    lax.fori_loop(1, NCL // 2, pair, 0)
    for b in (0, 1):
        off2 = base + (NCL - 2 + b) * KL
        pltpu.make_async_copy(ru_w[b], gu_o.at[pl.ds(off2, KL)], s_w1[b]).wait()
        pltpu.make_async_copy(rm_w[b], gm_o.at[pl.ds(off2, KL)], s_w2[b]).wait()


_label_gather = pl.kernel(
    _label_gather_body,
    out_type=(jax.ShapeDtypeStruct((LPAD, D), _f32),
              jax.ShapeDtypeStruct((LPAD, D), _f32)),
    mesh=_mesh,
    scratch_types=[
        pltpu.VMEM((LPW,), jnp.int32),
        pltpu.VMEM((LPW,), jnp.int32),
        pltpu.VMEM((KL, D), _f32),
        pltpu.VMEM((KL, D), _f32),
        pltpu.VMEM((KL, D), _f32),
        pltpu.VMEM((KL, D), _f32),
    ] + [pltpu.SemaphoreType.DMA] * 8,
)


_BR = 512


def _sage_half(acc, cnt, x, Wl, b, Wr):
    s = acc[...]
    ctot = cnt[...]
    inv = 1.0 / jnp.maximum(ctot, 1.0)
    mean = s * inv[:, None]
    return (jnp.dot(mean, Wl[...], preferred_element_type=jnp.float32)
            + b[...]
            + jnp.dot(x[...], Wr[...], preferred_element_type=jnp.float32))


def _tc_layer1_body(acc_m, cnt_m, xm, WlA, bA, WrA,
                    acc_u, cnt_u, xu, WlB, bB, WrB, hm_o, hu_o):
    hm_o[...] = jnp.maximum(_sage_half(acc_m, cnt_m, xm, WlA, bA, WrA), 0.0)
    hu_o[...] = jnp.maximum(_sage_half(acc_u, cnt_u, xu, WlB, bB, WrB), 0.0)


def _tc_layer2_body(acc_m, cnt_m, xm, WlA, bA, WrA,
                    acc_u, cnt_u, xu, WlB, bB, WrB,
                    WpU, bpU, WpM, pu_o, pm_o):
    zm = _sage_half(acc_m, cnt_m, xm, WlA, bA, WrA)
    zu = _sage_half(acc_u, cnt_u, xu, WlB, bB, WrB)
    pu_o[...] = jnp.dot(zu, WpU[...], preferred_element_type=jnp.float32) + bpU[...]
    pm_o[...] = jnp.dot(zm, WpM[...], preferred_element_type=jnp.float32)


def _acc_spec():
    return pl.BlockSpec((_BR, D), lambda i: (i, 0))


def _cnt_spec():
    return pl.BlockSpec((_BR,), lambda i: (i,))


def _row_spec(w=D):
    return pl.BlockSpec((_BR, w), lambda i: (i, 0))


def _w_spec(w=D):
    return pl.BlockSpec((D, w), lambda i: (0, 0))


def _b_spec(w=D):
    return pl.BlockSpec((1, w), lambda i: (0, 0))


_tc_layer1 = pl.pallas_call(
    _tc_layer1_body,
    grid=(NPAD // _BR,),
    in_specs=[_acc_spec(), _cnt_spec(), _row_spec(), _w_spec(), _b_spec(), _w_spec(),
              _acc_spec(), _cnt_spec(), _row_spec(), _w_spec(), _b_spec(), _w_spec()],
    out_specs=(_row_spec(), _row_spec()),
    out_shape=(jax.ShapeDtypeStruct((NPAD, D), _f32),
               jax.ShapeDtypeStruct((NPAD, D), _f32)),
)

_tc_layer2 = pl.pallas_call(
    _tc_layer2_body,
    grid=(NPAD // _BR,),
    in_specs=[_acc_spec(), _cnt_spec(), _row_spec(), _w_spec(), _b_spec(), _w_spec(),
              _acc_spec(), _cnt_spec(), _row_spec(), _w_spec(), _b_spec(), _w_spec(),
              _w_spec(D), _b_spec(D), _w_spec(D)],
    out_specs=(_row_spec(D), _row_spec(D)),
    out_shape=(jax.ShapeDtypeStruct((NPAD, D), _f32),
               jax.ShapeDtypeStruct((NPAD, D), _f32)),
)


def _head_body(gu, gm, o):
    s = gu[...] + gm[...]
    colid = lax.broadcasted_iota(jnp.int32, s.shape, 1)
    o[...] = jnp.where(colid == 1, jax.nn.softplus(s) + 1e-6, s)


_BL = 1024
_head = pl.pallas_call(
    _head_body,
    grid=(LPAD // _BL,),
    in_specs=[pl.BlockSpec((_BL, D), lambda i: (i, 0)),
              pl.BlockSpec((_BL, D), lambda i: (i, 0))],
    out_specs=pl.BlockSpec((_BL, D), lambda i: (i, 0)),
    out_shape=jax.ShapeDtypeStruct((LPAD, D), _f32),
)


def kernel(x_user, x_movie, edge_index, edge_label_index,
           W1l_um, b1l_um, W1r_um, W1l_mu, b1l_mu, W1r_mu,
           W2l_um, b2l_um, W2r_um, W2l_mu, b2l_mu, W2r_mu, Wlin, blin):
    f32 = jnp.float32
    pad_n = NPAD - N
    xu = jnp.pad(x_user.astype(f32), ((0, pad_n), (0, 0)))
    xm = jnp.pad(x_movie.astype(f32), ((0, pad_n), (0, 0)))
    iu = edge_index[0].astype(jnp.int32)
    im = edge_index[1].astype(jnp.int32)
    row = jnp.pad(edge_label_index[0].astype(jnp.int32), (0, LPAD - L))
    col = jnp.pad(edge_label_index[1].astype(jnp.int32), (0, LPAD - L))

    zb = jnp.zeros((NPAD, D), f32)
    zs = jnp.zeros((NPAD,), f32)
    ones = jnp.ones((KE,), f32)

    acc_m, acc_u, cnt_m, cnt_u = _segsum_counts(xu, xm, iu, im, zb, zs, ones)

    b1um = b1l_um.reshape(1, D).astype(f32)
    b1mu = b1l_mu.reshape(1, D).astype(f32)
    h_movie, h_user = _tc_layer1(acc_m, cnt_m, xm, W1l_um, b1um, W1r_um,
                                 acc_u, cnt_u, xu, W1l_mu, b1mu, W1r_mu)

    acc2_m, acc2_u = _segsum(h_user, h_movie, iu, im, zb)

    WpU = jnp.zeros((D, D), f32).at[:, 0:2].set(Wlin[:D].astype(f32))
    WpM = jnp.zeros((D, D), f32).at[:, 0:2].set(Wlin[D:].astype(f32))
    bp = jnp.zeros((1, D), f32).at[0, 0:2].set(blin.astype(f32))
    b2um = b2l_um.reshape(1, D).astype(f32)
    b2mu = b2l_mu.reshape(1, D).astype(f32)
    p_user, p_movie = _tc_layer2(acc2_m, cnt_m, h_movie, W2l_um, b2um, W2r_um,
                                 acc2_u, cnt_u, h_user, W2l_mu, b2mu, W2r_mu,
                                 WpU, bp, WpM)

    gu, gm = _label_gather(p_user, p_movie, row, col)
    out = _head(gu, gm)
    return out[:L, 0], out[:L, 1]
